# Initial kernel scaffold; baseline (speedup 1.0000x reference)
import functools

import jax
import jax.numpy as jnp
from jax import lax
from jax.experimental import pallas as pl
from jax.experimental.pallas import tpu as pltpu
from jax.experimental.pallas import tpu_sc as plsc

_N = 10000
_E = 320000
_D = 128
_NW = 32
_CPW = _D // _NW
_CHUNK = 2000
_NCHUNK = _E // _CHUNK
_GROUPS = _CHUNK // 16
_NVEC = _N // 16

_mesh = plsc.VectorSubcoreMesh(core_axis_name="c", subcore_axis_name="s")


@functools.partial(
    pl.kernel,
    out_type=jax.ShapeDtypeStruct((_NW, 16), jnp.float32),
    mesh=_mesh,
    scratch_types=[
        pltpu.VMEM((_CPW, _N), jnp.float32),
        pltpu.VMEM((_CPW, _N), jnp.float32),
        pltpu.VMEM((_N,), jnp.float32),
        pltpu.VMEM((_CHUNK,), jnp.int32),
        pltpu.VMEM((_CHUNK,), jnp.int32),
        pltpu.VMEM((_CHUNK,), jnp.float32),
        pltpu.VMEM((16,), jnp.float32),
    ],
)
def _hyper_sc(xT, srcr, dstr, enr, nnr, out, xv, hv, nnv, sbuf, dbuf, ebuf, ov):
    wid = lax.axis_index("s") * 2 + lax.axis_index("c")

    pltpu.sync_copy(xT.at[pl.ds(wid * _CPW, _CPW), :], xv)
    pltpu.sync_copy(nnr, nnv)

    zeros = jnp.zeros((16,), jnp.float32)
    for c in range(_CPW):
        def zbody(i, _, c=c):
            hv[c, pl.ds(pl.multiple_of(i * 16, 16), 16)] = zeros
            return 0
        lax.fori_loop(0, _NVEC, zbody, 0)

    def chunk_body(ci, _):
        off = pl.multiple_of(ci * _CHUNK, 8)
        pltpu.sync_copy(srcr.at[pl.ds(off, _CHUNK)], sbuf)
        pltpu.sync_copy(dstr.at[pl.ds(off, _CHUNK)], dbuf)
        pltpu.sync_copy(enr.at[pl.ds(off, _CHUNK)], ebuf)

        def g_body(g, _):
            o = pl.multiple_of(g * 16, 16)
            s = sbuf[pl.ds(o, 16)]
            d = dbuf[pl.ds(o, 16)]
            e = ebuf[pl.ds(o, 16)]
            cn = plsc.load_gather(nnv, [s]) * plsc.load_gather(nnv, [d]) * e
            for c in range(_CPW):
                cc = jnp.full((16,), c, jnp.int32)
                v = plsc.load_gather(xv, [cc, s]) * cn
                v = jnp.maximum(v, 0.01 * v)
                plsc.addupdate_scatter(hv, [cc, d], v)
            return 0

        lax.fori_loop(0, _GROUPS, g_body, 0)
        return 0

    lax.fori_loop(0, _NCHUNK, chunk_body, 0)

    lanes = lax.iota(jnp.int32, 16)
    row = jnp.zeros((16,), jnp.float32)
    ninf = jnp.full((16,), -jnp.inf, jnp.float32)
    for c in range(_CPW):
        def hbody(i, acc, c=c):
            return jnp.maximum(acc, hv[c, pl.ds(pl.multiple_of(i * 16, 16), 16)])
        def xbody(i, acc, c=c):
            return jnp.maximum(acc, xv[c, pl.ds(pl.multiple_of(i * 16, 16), 16)])
        hm = jnp.max(lax.fori_loop(0, _NVEC, hbody, ninf))
        xm = jnp.max(lax.fori_loop(0, _NVEC, xbody, ninf))
        row = jnp.where(lanes == c, hm, row)
        row = jnp.where(lanes == 8 + c, xm, row)
    ov[...] = row
    pltpu.sync_copy(ov, out.at[wid])


def kernel(x, edge_index, node_norm, edge_norm):
    xT = jnp.transpose(x)
    src = edge_index[0]
    dst = edge_index[1]
    rows = _hyper_sc(xT, src, dst, edge_norm, node_norm)
    hmax = rows[:, :_CPW].reshape(_D)
    xmax = rows[:, 8:8 + _CPW].reshape(_D)
    return (0.5 * (hmax + xmax))[None, :]


# speedup vs baseline: 5.5041x; 5.5041x over previous
import functools

import jax
import jax.numpy as jnp
from jax import lax
from jax.experimental import pallas as pl
from jax.experimental.pallas import tpu as pltpu
from jax.experimental.pallas import tpu_sc as plsc

_N = 10000
_E = 320000
_D = 128
_NW = 32
_CPW = _D // _NW
_CHUNK = 2000
_NCHUNK = _E // _CHUNK
_GROUPS = _CHUNK // 16
_NVEC = _N // 16

_mesh = plsc.VectorSubcoreMesh(core_axis_name="c", subcore_axis_name="s")


@functools.partial(
    pl.kernel,
    out_type=jax.ShapeDtypeStruct((_NW, 16), jnp.float32),
    mesh=_mesh,
    compiler_params=pltpu.CompilerParams(needs_layout_passes=False),
    scratch_types=[
        pltpu.VMEM((_CPW, _N), jnp.float32),
        pltpu.VMEM((_CPW, _N), jnp.float32),
        pltpu.VMEM((_N,), jnp.float32),
        pltpu.VMEM((_CHUNK,), jnp.int32),
        pltpu.VMEM((_CHUNK,), jnp.int32),
        pltpu.VMEM((_CHUNK,), jnp.float32),
        pltpu.VMEM((16,), jnp.float32),
    ],
)
def _hyper_sc(xT, srcr, dstr, enr, nnr, out, xv, hv, nnv, sbuf, dbuf, ebuf, ov):
    wid = lax.axis_index("s") * 2 + lax.axis_index("c")

    pltpu.sync_copy(xT.at[pl.ds(wid * _CPW, _CPW), :], xv)
    pltpu.sync_copy(nnr, nnv)

    zeros = jnp.zeros((16,), jnp.float32)
    for c in range(_CPW):
        def zbody(i, _, c=c):
            hv[c, pl.ds(pl.multiple_of(i * 16, 16), 16)] = zeros
            return 0
        lax.fori_loop(0, _NVEC, zbody, 0)

    def chunk_body(ci, _):
        off = pl.multiple_of(ci * _CHUNK, 8)
        pltpu.sync_copy(srcr.at[pl.ds(off, _CHUNK)], sbuf)
        pltpu.sync_copy(dstr.at[pl.ds(off, _CHUNK)], dbuf)
        pltpu.sync_copy(enr.at[pl.ds(off, _CHUNK)], ebuf)

        def g_body(g, _):
            o = pl.multiple_of(g * 16, 16)
            s = sbuf[pl.ds(o, 16)]
            d = dbuf[pl.ds(o, 16)]
            e = ebuf[pl.ds(o, 16)]
            cn = plsc.load_gather(nnv, [s]) * plsc.load_gather(nnv, [d]) * e
            for c in range(_CPW):
                cc = jnp.full((16,), c, jnp.int32)
                v = plsc.load_gather(xv, [cc, s]) * cn
                v = jnp.maximum(v, 0.01 * v)
                plsc.addupdate_scatter(hv, [cc, d], v)
            return 0

        lax.fori_loop(0, _GROUPS, g_body, 0)
        return 0

    lax.fori_loop(0, _NCHUNK, chunk_body, 0)

    lanes = lax.iota(jnp.int32, 16)
    row = jnp.zeros((16,), jnp.float32)
    ninf = jnp.full((16,), -jnp.inf, jnp.float32)
    for c in range(_CPW):
        def hbody(i, acc, c=c):
            return jnp.maximum(acc, hv[c, pl.ds(pl.multiple_of(i * 16, 16), 16)])
        def xbody(i, acc, c=c):
            return jnp.maximum(acc, xv[c, pl.ds(pl.multiple_of(i * 16, 16), 16)])
        hm = jnp.max(lax.fori_loop(0, _NVEC, hbody, ninf))
        xm = jnp.max(lax.fori_loop(0, _NVEC, xbody, ninf))
        row = jnp.where(lanes == c, hm, row)
        row = jnp.where(lanes == 8 + c, xm, row)
    ov[...] = row
    pltpu.sync_copy(ov, out.at[wid])


def kernel(x, edge_index, node_norm, edge_norm):
    xT = jnp.transpose(x)
    src = edge_index[0]
    dst = edge_index[1]
    rows = _hyper_sc(xT, src, dst, edge_norm, node_norm)
    hmax = rows[:, :_CPW].reshape(_D)
    xmax = rows[:, 8:8 + _CPW].reshape(_D)
    return (0.5 * (hmax + xmax))[None, :]


# speedup vs baseline: 18.6443x; 3.3874x over previous
import functools

import jax
import jax.numpy as jnp
from jax import lax
from jax.experimental import pallas as pl
from jax.experimental.pallas import tpu as pltpu
from jax.experimental.pallas import tpu_sc as plsc

_N = 10000
_E = 320000
_D = 128
_DH = _D // 2
_NW = 32
_EPW = _E // _NW
_EPWP = 10240
_EPAD = _NW * _EPWP
_PCHUNK = 2000
_PGROUPS = _PCHUNK // 16
_PCH = _EPW // _PCHUNK
_NPAD = _EPWP - _EPW
_C = 128
_EPT = _EPAD // 16
_CPT = _EPT // _C
_WIN = _CPT // 2
_HZR = 40
_NHZ = _N // _HZR

_mesh = plsc.VectorSubcoreMesh(core_axis_name="c", subcore_axis_name="s")
_params = pltpu.CompilerParams(
    needs_layout_passes=False, use_tc_tiling_on_sc=False
)


@functools.partial(
    pl.kernel,
    out_type=(
        jax.ShapeDtypeStruct((_EPAD,), jnp.int32),
        jax.ShapeDtypeStruct((_EPAD,), jnp.int32),
        jax.ShapeDtypeStruct((_EPAD,), jnp.float32),
    ),
    mesh=_mesh,
    compiler_params=_params,
    scratch_types=[
        pltpu.VMEM((_N,), jnp.float32),
        pltpu.VMEM((_PCHUNK,), jnp.int32),
        pltpu.VMEM((_PCHUNK,), jnp.int32),
        pltpu.VMEM((_PCHUNK,), jnp.float32),
        pltpu.VMEM((_PCHUNK,), jnp.float32),
        pltpu.VMEM((_NPAD,), jnp.int32),
        pltpu.VMEM((_NPAD,), jnp.float32),
    ],
)
def _prep_sc(srcr, dstr, enr, nnr, srcp, dstp, cnp, nnv, sbuf, dbuf, ebuf,
             ocn, zpi, zpf):
    wid = lax.axis_index("s") * 2 + lax.axis_index("c")
    pltpu.sync_copy(nnr, nnv)
    ibase = pl.multiple_of(wid * _EPW, 8)
    obase = pl.multiple_of(wid * _EPWP, 8)

    def chunk_body(k, _):
        ioff = pl.multiple_of(ibase + k * _PCHUNK, 8)
        ooff = pl.multiple_of(obase + k * _PCHUNK, 8)
        pltpu.sync_copy(srcr.at[pl.ds(ioff, _PCHUNK)], sbuf)
        pltpu.sync_copy(dstr.at[pl.ds(ioff, _PCHUNK)], dbuf)
        pltpu.sync_copy(enr.at[pl.ds(ioff, _PCHUNK)], ebuf)

        @plsc.parallel_loop(0, _PGROUPS, 1, unroll=2)
        def g_body(g):
            o = pl.multiple_of(g * 16, 16)
            s = sbuf[pl.ds(o, 16)]
            d = dbuf[pl.ds(o, 16)]
            e = ebuf[pl.ds(o, 16)]
            ocn[pl.ds(o, 16)] = (
                plsc.load_gather(nnv, [s]) * plsc.load_gather(nnv, [d]) * e
            )

        pltpu.sync_copy(sbuf, srcp.at[pl.ds(ooff, _PCHUNK)])
        pltpu.sync_copy(dbuf, dstp.at[pl.ds(ooff, _PCHUNK)])
        pltpu.sync_copy(ocn, cnp.at[pl.ds(ooff, _PCHUNK)])
        return 0

    lax.fori_loop(0, _PCH, chunk_body, 0)

    zi = jnp.zeros((16,), jnp.int32)
    zf = jnp.zeros((16,), jnp.float32)

    def zbody(i, _):
        o = pl.multiple_of(i * 16, 16)
        zpi[pl.ds(o, 16)] = zi
        zpf[pl.ds(o, 16)] = zf
        return 0

    lax.fori_loop(0, _NPAD // 16, zbody, 0)
    poff = pl.multiple_of(obase + _EPW, 8)
    pltpu.sync_copy(zpi, srcp.at[pl.ds(poff, _NPAD)])
    pltpu.sync_copy(zpi, dstp.at[pl.ds(poff, _NPAD)])
    pltpu.sync_copy(zpf, cnp.at[pl.ds(poff, _NPAD)])


@functools.partial(
    pl.kernel,
    out_type=jax.ShapeDtypeStruct((2, 2, _DH), jnp.float32),
    mesh=_mesh,
    compiler_params=_params,
    scratch_types=[
        pltpu.VMEM((_WIN, _C), jnp.int32),
        pltpu.VMEM((_WIN, _C), jnp.int32),
        pltpu.VMEM((_WIN * _C,), jnp.float32),
        pltpu.VMEM((3, _C, _DH), jnp.float32),
        pltpu.VMEM((2, _DH), jnp.float32),
        pltpu.VMEM((16, 2, _DH), jnp.float32),
        pltpu.VMEM_SHARED((_N, _DH), jnp.float32),
        pltpu.VMEM_SHARED((16, 2, _DH), jnp.float32),
        pltpu.SemaphoreType.DMA,
        pltpu.SemaphoreType.DMA,
    ],
)
def _scatter_sc(xi, src2, dst2, cnp, out, sv, dv, cv, msg, pbuf, gbuf,
                hsh, shr, semg, sems):
    cid = lax.axis_index("c")
    sid = lax.axis_index("s")
    zf = jnp.zeros((16,), jnp.float32)

    def zrow(r, _):
        for k in range(_DH // 16):
            msg[0, r, pl.ds(k * 16, 16)] = zf
        return 0

    lax.fori_loop(0, _HZR, zrow, 0)

    def hz_body(k, _):
        j = sid + k * 16

        @pl.when(j < _NHZ)
        def _():
            o = pl.multiple_of(j * _HZR, 8)
            pltpu.sync_copy(msg.at[0, pl.ds(0, _HZR), :],
                            hsh.at[pl.ds(o, _HZR), :])

        return 0

    lax.fori_loop(0, (_NHZ + 15) // 16, hz_body, 0)
    plsc.subcore_barrier()

    def wait_one(sem):
        pltpu.make_async_copy(xi.at[pl.ds(0, _C), :], msg.at[0], sem).wait()

    rbase = sid * _CPT

    for wnd in range(2):
        roff = pl.multiple_of(rbase + wnd * _WIN, 8)
        eoff = pl.multiple_of((sid * _CPT + wnd * _WIN) * _C, 8)
        pltpu.sync_copy(src2.at[pl.ds(roff, _WIN), :], sv)
        pltpu.sync_copy(dst2.at[pl.ds(roff, _WIN), :], dv)
        pltpu.sync_copy(cnp.at[pl.ds(eoff, _WIN * _C)], cv)
        coff = cid * _N

        def obody(r, _):
            for k in range(_C // 16):
                sl = pl.ds(k * 16, 16)
                sv[r, sl] = sv[r, sl] + coff
            return 0

        lax.fori_loop(0, _WIN, obody, 0)

        pltpu.async_copy(xi.at[sv.at[0]], msg.at[0], semg)

        def chunk_body(j, _):
            slot = lax.rem(j, 3)

            @pl.when(j + 1 < _WIN)
            def _():
                @pl.when(j >= 2)
                def _():
                    wait_one(sems)

                pltpu.async_copy(
                    xi.at[sv.at[j + 1]], msg.at[lax.rem(j + 1, 3)], semg
                )

            wait_one(semg)
            ebase = j * _C

            @plsc.parallel_loop(0, _C, 1, unroll=2)
            def ebody(e):
                ce = plsc.load_gather(
                    cv, [jnp.full((16,), ebase + e, jnp.int32)]
                )
                for k in range(_DH // 16):
                    v = msg[slot, e, pl.ds(k * 16, 16)] * ce
                    v = jnp.maximum(v, 0.01 * v)
                    msg[slot, e, pl.ds(k * 16, 16)] = v

            pltpu.async_copy(msg.at[slot], hsh.at[dv.at[j]], sems, add=True)
            return 0

        lax.fori_loop(0, _WIN, chunk_body, 0)
        wait_one(sems)
        wait_one(sems)
        wait_one(sems)

    plsc.subcore_barrier()

    ninf = jnp.full((16,), -jnp.inf, jnp.float32)
    nacc = _DH // 16

    def red_body(k, acc):
        j = sid + k * 16

        def do(acc):
            o = pl.multiple_of(j * _HZR, 8)
            pltpu.sync_copy(hsh.at[pl.ds(o, _HZR), :],
                            msg.at[0, pl.ds(0, _HZR), :])
            xo = pl.multiple_of(cid * _N + j * _HZR, 8)
            pltpu.sync_copy(xi.at[pl.ds(xo, _HZR), :],
                            msg.at[1, pl.ds(0, _HZR), :])

            def row(r, acc):
                na = []
                for k2 in range(nacc):
                    na.append(
                        jnp.maximum(acc[k2], msg[0, r, pl.ds(k2 * 16, 16)])
                    )
                for k2 in range(nacc):
                    na.append(
                        jnp.maximum(acc[nacc + k2],
                                    msg[1, r, pl.ds(k2 * 16, 16)])
                    )
                return tuple(na)

            return lax.fori_loop(0, _HZR, row, acc)

        return lax.cond(j < _NHZ, do, lambda a: a, acc)

    acc = lax.fori_loop(0, (_NHZ + 15) // 16, red_body, (ninf,) * (2 * nacc))

    for k in range(nacc):
        pbuf[0, pl.ds(k * 16, 16)] = acc[k]
        pbuf[1, pl.ds(k * 16, 16)] = acc[nacc + k]
    pltpu.sync_copy(pbuf, shr.at[sid])
    plsc.subcore_barrier()

    @pl.when(sid == 0)
    def _():
        pltpu.sync_copy(shr, gbuf)

        def tmax(t, acc):
            na = []
            for k in range(nacc):
                na.append(jnp.maximum(acc[k], gbuf[t, 0, pl.ds(k * 16, 16)]))
            for k in range(nacc):
                na.append(
                    jnp.maximum(acc[nacc + k], gbuf[t, 1, pl.ds(k * 16, 16)])
                )
            return tuple(na)

        facc = lax.fori_loop(0, 16, tmax, (ninf,) * (2 * nacc))
        for k in range(nacc):
            pbuf[0, pl.ds(k * 16, 16)] = facc[k]
            pbuf[1, pl.ds(k * 16, 16)] = facc[nacc + k]
        pltpu.sync_copy(pbuf, out.at[cid])


def kernel(x, edge_index, node_norm, edge_norm):
    src = edge_index[0]
    dst = edge_index[1]
    srcp, dstp, cnp = _prep_sc(src, dst, edge_norm, node_norm)
    src2 = srcp.reshape(_EPAD // _C, _C)
    dst2 = dstp.reshape(_EPAD // _C, _C)
    xi = jnp.concatenate([x[:, :_DH], x[:, _DH:]], axis=0)
    red = _scatter_sc(xi, src2, dst2, cnp)
    hmax = jnp.concatenate([red[0, 0], red[1, 0]])
    xmax = jnp.concatenate([red[0, 1], red[1, 1]])
    return (0.5 * (hmax + xmax))[None, :]


# speedup vs baseline: 23.0568x; 1.2367x over previous
import functools

import jax
import jax.numpy as jnp
from jax import lax
from jax.experimental import pallas as pl
from jax.experimental.pallas import tpu as pltpu
from jax.experimental.pallas import tpu_sc as plsc

_N = 10000
_E = 320000
_D = 128
_NW = 32
_CPW = _D // _NW
_EPW = _E // _NW
_PCHUNK = 2000
_PGROUPS = _PCHUNK // 16
_PCH = _EPW // _PCHUNK
_CHUNK = 8000
_NCHUNK = _E // _CHUNK
_GROUPS = _CHUNK // 16
_NVEC = _N // 16

_mesh = plsc.VectorSubcoreMesh(core_axis_name="c", subcore_axis_name="s")
_params = pltpu.CompilerParams(needs_layout_passes=False)


@functools.partial(
    pl.kernel,
    out_type=(
        jax.ShapeDtypeStruct((_E,), jnp.int32),
        jax.ShapeDtypeStruct((_E,), jnp.float32),
    ),
    mesh=_mesh,
    compiler_params=_params,
    scratch_types=[
        pltpu.VMEM((_N,), jnp.float32),
        pltpu.VMEM((_PCHUNK,), jnp.int32),
        pltpu.VMEM((_PCHUNK,), jnp.int32),
        pltpu.VMEM((_PCHUNK,), jnp.float32),
        pltpu.VMEM((_PCHUNK,), jnp.int32),
        pltpu.VMEM((_PCHUNK,), jnp.float32),
    ],
)
def _prep_sc(srcr, dstr, enr, nnr, pkr, cnr, nnv, sbuf, dbuf, ebuf, opk, ocn):
    wid = lax.axis_index("s") * 2 + lax.axis_index("c")
    pltpu.sync_copy(nnr, nnv)
    base = pl.multiple_of(wid * _EPW, 8)

    def chunk_body(k, _):
        off = pl.multiple_of(base + k * _PCHUNK, 8)
        pltpu.sync_copy(srcr.at[pl.ds(off, _PCHUNK)], sbuf)
        pltpu.sync_copy(dstr.at[pl.ds(off, _PCHUNK)], dbuf)
        pltpu.sync_copy(enr.at[pl.ds(off, _PCHUNK)], ebuf)

        @plsc.parallel_loop(0, _PGROUPS, 1, unroll=2)
        def g_body(g):
            o = pl.multiple_of(g * 16, 16)
            s = sbuf[pl.ds(o, 16)]
            d = dbuf[pl.ds(o, 16)]
            e = ebuf[pl.ds(o, 16)]
            cn = plsc.load_gather(nnv, [s]) * plsc.load_gather(nnv, [d]) * e
            opk[pl.ds(o, 16)] = jnp.bitwise_or(s, jnp.left_shift(d, 16))
            ocn[pl.ds(o, 16)] = cn

        pltpu.sync_copy(opk, pkr.at[pl.ds(off, _PCHUNK)])
        pltpu.sync_copy(ocn, cnr.at[pl.ds(off, _PCHUNK)])
        return 0

    lax.fori_loop(0, _PCH, chunk_body, 0)


@functools.partial(
    pl.kernel,
    out_type=jax.ShapeDtypeStruct((_NW, 16), jnp.float32),
    mesh=_mesh,
    compiler_params=_params,
    scratch_types=[
        pltpu.VMEM((_CPW * _N,), jnp.float32),
        pltpu.VMEM((_CPW * _N,), jnp.float32),
        pltpu.VMEM((2 * _CHUNK,), jnp.int32),
        pltpu.VMEM((2 * _CHUNK,), jnp.float32),
        pltpu.VMEM((16,), jnp.float32),
        pltpu.SemaphoreType.DMA,
    ],
)
def _main_sc(xT, pkr, cnr, out, xv, hv, pbuf, cbuf, ov, sem):
    wid = lax.axis_index("s") * 2 + lax.axis_index("c")

    for c in range(_CPW):
        off = pl.multiple_of((wid * _CPW + c) * _N, 8)
        pltpu.sync_copy(xT.at[pl.ds(off, _N)], xv.at[pl.ds(c * _N, _N)])

    zeros = jnp.zeros((16,), jnp.float32)

    def zbody(i, _):
        hv[pl.ds(pl.multiple_of(i * 16, 16), 16)] = zeros
        return 0

    lax.fori_loop(0, _CPW * _NVEC, zbody, 0)

    pltpu.async_copy(pkr.at[pl.ds(0, _CHUNK)], pbuf.at[pl.ds(0, _CHUNK)], sem)
    pltpu.async_copy(cnr.at[pl.ds(0, _CHUNK)], cbuf.at[pl.ds(0, _CHUNK)], sem)

    def chunk_body(ci, _):
        sbase = pl.multiple_of(lax.rem(ci, 2) * _CHUNK, 8)
        pltpu.make_async_copy(
            pkr.at[pl.ds(0, _CHUNK)], pbuf.at[pl.ds(sbase, _CHUNK)], sem
        ).wait()
        pltpu.make_async_copy(
            cnr.at[pl.ds(0, _CHUNK)], cbuf.at[pl.ds(sbase, _CHUNK)], sem
        ).wait()

        @pl.when(ci + 1 < _NCHUNK)
        def _():
            nbase = pl.multiple_of(lax.rem(ci + 1, 2) * _CHUNK, 8)
            noff = pl.multiple_of((ci + 1) * _CHUNK, 8)
            pltpu.async_copy(
                pkr.at[pl.ds(noff, _CHUNK)], pbuf.at[pl.ds(nbase, _CHUNK)], sem
            )
            pltpu.async_copy(
                cnr.at[pl.ds(noff, _CHUNK)], cbuf.at[pl.ds(nbase, _CHUNK)], sem
            )

        @plsc.parallel_loop(0, _GROUPS, 1, unroll=4)
        def g_body(g):
            o = pl.multiple_of(sbase + g * 16, 16)
            p = pbuf[pl.ds(o, 16)]
            cn = cbuf[pl.ds(o, 16)]
            s = jnp.bitwise_and(p, 0xFFFF)
            d = lax.shift_right_logical(p, 16)
            for c in range(_CPW):
                vs = s + (c * _N) if c else s
                vd = d + (c * _N) if c else d
                v = plsc.load_gather(xv, [vs]) * cn
                v = jnp.maximum(v, 0.01 * v)
                plsc.addupdate_scatter(hv, [vd], v)

        return 0

    lax.fori_loop(0, _NCHUNK, chunk_body, 0)

    lanes = lax.iota(jnp.int32, 16)
    row = jnp.zeros((16,), jnp.float32)
    ninf = jnp.full((16,), -jnp.inf, jnp.float32)
    for c in range(_CPW):
        def hbody(i, acc, c=c):
            o = pl.multiple_of(c * _N + i * 16, 16)
            return jnp.maximum(acc, hv[pl.ds(o, 16)])

        def xbody(i, acc, c=c):
            o = pl.multiple_of(c * _N + i * 16, 16)
            return jnp.maximum(acc, xv[pl.ds(o, 16)])

        hm = jnp.max(lax.fori_loop(0, _NVEC, hbody, ninf))
        xm = jnp.max(lax.fori_loop(0, _NVEC, xbody, ninf))
        row = jnp.where(lanes == c, hm, row)
        row = jnp.where(lanes == 8 + c, xm, row)
    ov[...] = row
    pltpu.sync_copy(ov, out.at[wid])


def kernel(x, edge_index, node_norm, edge_norm):
    xT = jnp.transpose(x).reshape(-1)
    src = edge_index[0]
    dst = edge_index[1]
    pk, cn = _prep_sc(src, dst, edge_norm, node_norm)
    rows = _main_sc(xT, pk, cn)
    hmax = rows[:, :_CPW].reshape(_D)
    xmax = rows[:, 8:8 + _CPW].reshape(_D)
    return (0.5 * (hmax + xmax))[None, :]


# speedup vs baseline: 23.2519x; 1.0085x over previous
import functools

import jax
import jax.numpy as jnp
from jax import lax
from jax.experimental import pallas as pl
from jax.experimental.pallas import tpu as pltpu
from jax.experimental.pallas import tpu_sc as plsc

_N = 10000
_E = 320000
_D = 128
_NW = 32
_CPW = _D // _NW
_EPW = _E // _NW
_PCHUNK = 2000
_PGROUPS = _PCHUNK // 16
_PCH = _EPW // _PCHUNK
_CHUNK = 8000
_NCHUNK = _E // _CHUNK
_GROUPS = _CHUNK // 16
_NVEC = _N // 16

_mesh = plsc.VectorSubcoreMesh(core_axis_name="c", subcore_axis_name="s")
_params = pltpu.CompilerParams(needs_layout_passes=False)


@functools.partial(
    pl.kernel,
    out_type=(
        jax.ShapeDtypeStruct((_E,), jnp.int32),
        jax.ShapeDtypeStruct((_E,), jnp.float32),
    ),
    mesh=_mesh,
    compiler_params=_params,
    scratch_types=[
        pltpu.VMEM((_N,), jnp.float32),
        pltpu.VMEM((_PCHUNK,), jnp.int32),
        pltpu.VMEM((_PCHUNK,), jnp.int32),
        pltpu.VMEM((_PCHUNK,), jnp.float32),
        pltpu.VMEM((_PCHUNK,), jnp.int32),
        pltpu.VMEM((_PCHUNK,), jnp.float32),
    ],
)
def _prep_sc(srcr, dstr, enr, nnr, pkr, cnr, nnv, sbuf, dbuf, ebuf, opk, ocn):
    wid = lax.axis_index("s") * 2 + lax.axis_index("c")
    pltpu.sync_copy(nnr, nnv)
    base = pl.multiple_of(wid * _EPW, 8)

    def chunk_body(k, _):
        off = pl.multiple_of(base + k * _PCHUNK, 8)
        pltpu.sync_copy(srcr.at[pl.ds(off, _PCHUNK)], sbuf)
        pltpu.sync_copy(dstr.at[pl.ds(off, _PCHUNK)], dbuf)
        pltpu.sync_copy(enr.at[pl.ds(off, _PCHUNK)], ebuf)

        @plsc.parallel_loop(0, _PGROUPS, 1, unroll=2)
        def g_body(g):
            o = pl.multiple_of(g * 16, 16)
            s = sbuf[pl.ds(o, 16)]
            d = dbuf[pl.ds(o, 16)]
            e = ebuf[pl.ds(o, 16)]
            cn = plsc.load_gather(nnv, [s]) * plsc.load_gather(nnv, [d]) * e
            opk[pl.ds(o, 16)] = jnp.bitwise_or(s, jnp.left_shift(d, 16))
            ocn[pl.ds(o, 16)] = cn

        pltpu.sync_copy(opk, pkr.at[pl.ds(off, _PCHUNK)])
        pltpu.sync_copy(ocn, cnr.at[pl.ds(off, _PCHUNK)])
        return 0

    lax.fori_loop(0, _PCH, chunk_body, 0)


@functools.partial(
    pl.kernel,
    out_type=jax.ShapeDtypeStruct((_NW, 16), jnp.float32),
    mesh=_mesh,
    compiler_params=_params,
    scratch_types=[
        pltpu.VMEM((_CPW * _N,), jnp.float32),
        pltpu.VMEM((_CPW * _N,), jnp.float32),
        pltpu.VMEM((2 * _CHUNK,), jnp.int32),
        pltpu.VMEM((2 * _CHUNK,), jnp.float32),
        pltpu.VMEM((16,), jnp.float32),
        pltpu.SemaphoreType.DMA,
    ],
)
def _main_sc(xT, pkr, cnr, out, xv, hv, pbuf, cbuf, ov, sem):
    wid = lax.axis_index("s") * 2 + lax.axis_index("c")

    for c in range(_CPW):
        off = pl.multiple_of((wid * _CPW + c) * _N, 8)
        pltpu.sync_copy(xT.at[pl.ds(off, _N)], xv.at[pl.ds(c * _N, _N)])

    zeros = jnp.zeros((16,), jnp.float32)

    def zbody(i, _):
        hv[pl.ds(pl.multiple_of(i * 16, 16), 16)] = zeros
        return 0

    lax.fori_loop(0, _CPW * _NVEC, zbody, 0)

    pltpu.async_copy(pkr.at[pl.ds(0, _CHUNK)], pbuf.at[pl.ds(0, _CHUNK)], sem)
    pltpu.async_copy(cnr.at[pl.ds(0, _CHUNK)], cbuf.at[pl.ds(0, _CHUNK)], sem)

    def chunk_body(ci, _):
        sbase = pl.multiple_of(lax.rem(ci, 2) * _CHUNK, 8)
        pltpu.make_async_copy(
            pkr.at[pl.ds(0, _CHUNK)], pbuf.at[pl.ds(sbase, _CHUNK)], sem
        ).wait()
        pltpu.make_async_copy(
            cnr.at[pl.ds(0, _CHUNK)], cbuf.at[pl.ds(sbase, _CHUNK)], sem
        ).wait()

        @pl.when(ci + 1 < _NCHUNK)
        def _():
            nbase = pl.multiple_of(lax.rem(ci + 1, 2) * _CHUNK, 8)
            noff = pl.multiple_of((ci + 1) * _CHUNK, 8)
            pltpu.async_copy(
                pkr.at[pl.ds(noff, _CHUNK)], pbuf.at[pl.ds(nbase, _CHUNK)], sem
            )
            pltpu.async_copy(
                cnr.at[pl.ds(noff, _CHUNK)], cbuf.at[pl.ds(nbase, _CHUNK)], sem
            )

        @plsc.parallel_loop(0, _GROUPS, 1, unroll=2)
        def g_body(g):
            o = pl.multiple_of(sbase + g * 16, 16)
            p = pbuf[pl.ds(o, 16)]
            cn = cbuf[pl.ds(o, 16)]
            s = jnp.bitwise_and(p, 0xFFFF)
            d = lax.shift_right_logical(p, 16)
            for c in range(_CPW):
                vs = s + (c * _N) if c else s
                vd = d + (c * _N) if c else d
                v = plsc.load_gather(xv, [vs]) * cn
                v = jnp.maximum(v, 0.01 * v)
                plsc.addupdate_scatter(hv, [vd], v)

        return 0

    lax.fori_loop(0, _NCHUNK, chunk_body, 0)

    lanes = lax.iota(jnp.int32, 16)
    row = jnp.zeros((16,), jnp.float32)
    ninf = jnp.full((16,), -jnp.inf, jnp.float32)
    for c in range(_CPW):
        def hbody(i, acc, c=c):
            o = pl.multiple_of(c * _N + i * 16, 16)
            return jnp.maximum(acc, hv[pl.ds(o, 16)])

        def xbody(i, acc, c=c):
            o = pl.multiple_of(c * _N + i * 16, 16)
            return jnp.maximum(acc, xv[pl.ds(o, 16)])

        hm = jnp.max(lax.fori_loop(0, _NVEC, hbody, ninf))
        xm = jnp.max(lax.fori_loop(0, _NVEC, xbody, ninf))
        row = jnp.where(lanes == c, hm, row)
        row = jnp.where(lanes == 8 + c, xm, row)
    ov[...] = row
    pltpu.sync_copy(ov, out.at[wid])


def kernel(x, edge_index, node_norm, edge_norm):
    xT = jnp.transpose(x).reshape(-1)
    src = edge_index[0]
    dst = edge_index[1]
    pk, cn = _prep_sc(src, dst, edge_norm, node_norm)
    rows = _main_sc(xT, pk, cn)
    hmax = rows[:, :_CPW].reshape(_D)
    xmax = rows[:, 8:8 + _CPW].reshape(_D)
    return (0.5 * (hmax + xmax))[None, :]
